# baseline (speedup 1.0000x reference)
import jax
import jax.numpy as jnp
from jax import lax
from jax.experimental import pallas as pl
from jax.experimental.pallas import tpu as pltpu
from jax.experimental.pallas import tpu_sc as plsc

N = 10000
E = 320000
DF = 128
DE = 16
HD = 128
G = 64

NP = 10240
NC, NS, L = 2, 16, 16
NW = NC * NS
CHUNK = 128
EPT = 10240
EP = NW * EPT
NCHK = EPT // CHUNK
RPT = NP // NS
F32 = jnp.float32


def _sc_edge_body(z_hbm, c_hbm, idx_hbm, out_hbm,
                  idxA, idxB, bufA, bufB, agg_sh,
                  isA, isB, csA, csB, gsA, gsB, ssA, ssB):
    c = lax.axis_index("c")
    s = lax.axis_index("s")
    tile = c * NS + s
    ebase = tile * EPT
    idxs = (idxA, idxB)
    bufs = (bufA, bufB)
    iss = (isA, isB)
    css = (csA, csB)
    gss = (gsA, gsB)
    sss = (ssA, ssB)

    def zset(i, carry):
        for k in range(HD // L):
            bufA[i, pl.ds(k * L, L)] = jnp.zeros((L,), F32)
        return carry
    lax.fori_loop(0, CHUNK, zset, 0)
    r0 = s * RPT
    for q in range(RPT // CHUNK):
        pltpu.sync_copy(bufA, agg_sh.at[pl.ds(r0 + q * CHUNK, CHUNK)])
    plsc.subcore_barrier()

    def ixissue(j, b):
        pltpu.async_copy(idx_hbm.at[tile, j], idxs[b], iss[b])

    def ixwait(b):
        pltpu.make_async_copy(idx_hbm.at[tile, 0], idxs[b], iss[b]).wait()

    def cissue(j, b):
        pltpu.async_copy(c_hbm.at[pl.ds(ebase + j * CHUNK, CHUNK)],
                         bufs[b], css[b])

    def cwait(b):
        pltpu.make_async_copy(c_hbm.at[pl.ds(ebase, CHUNK)],
                              bufs[b], css[b]).wait()

    def gissue(b):
        pltpu.async_copy(z_hbm.at[idxs[b].at[0]], bufs[b], gss[b], add=True)

    def gwait(b):
        pltpu.make_async_copy(z_hbm.at[idxs[b].at[0]], bufs[b], gss[b]).wait()

    def sissue(b):
        pltpu.async_copy(bufs[b], agg_sh.at[idxs[b].at[1]], sss[b], add=True)

    def swait(b):
        pltpu.make_async_copy(bufs[b], agg_sh.at[idxs[b].at[1]], sss[b]).wait()

    def relu_buf(buf):
        def erow(e, cc):
            for k in range(HD // L):
                sl = pl.ds(k * L, L)
                buf[e, sl] = jnp.maximum(buf[e, sl], 0.0)
            return cc
        lax.fori_loop(0, CHUNK, erow, 0)

    ixissue(0, 0)
    cissue(0, 0)
    ixwait(0)
    cwait(0)
    gissue(0)

    def pair_body(i, carry):
        for b in (0, 1):
            j = 2 * i + b
            o = 1 - b
            gwait(b)
            relu_buf(bufs[b])
            sissue(b)

            @pl.when(j + 1 < NCHK)
            def _():
                @pl.when(j >= 1)
                def _():
                    swait(o)
                ixissue(j + 1, o)
                cissue(j + 1, o)
                ixwait(o)
                cwait(o)
                gissue(o)
        return carry
    PAIRS = (NCHK - 1) // 2
    lax.fori_loop(0, PAIRS, pair_body, 0)

    for j in range(2 * PAIRS, NCHK):
        b = j % 2
        o = 1 - b
        gwait(b)
        relu_buf(bufs[b])
        pltpu.sync_copy(bufs[b], agg_sh.at[idxs[b].at[1]], add=True)
        if j + 1 < NCHK:
            swait(o)
            ixissue(j + 1, o)
            cissue(j + 1, o)
            ixwait(o)
            cwait(o)
            gissue(o)
        elif 0 <= j - 1 < 2 * PAIRS:
            swait(o)
    plsc.subcore_barrier()

    for q in range(RPT // CHUNK):
        rr = r0 + q * CHUNK
        pltpu.sync_copy(agg_sh.at[pl.ds(rr, CHUNK)], bufA)
        pltpu.sync_copy(bufA, out_hbm.at[c, pl.ds(rr, CHUNK)])


_sc_edge_pass = pl.kernel(
    _sc_edge_body,
    out_type=jax.ShapeDtypeStruct((NC, NP, HD), F32),
    mesh=plsc.VectorSubcoreMesh(core_axis_name="c", subcore_axis_name="s",
                                num_cores=NC, num_subcores=NS),
    scratch_types=(
        [pltpu.VMEM((2, CHUNK), jnp.int32) for _ in range(2)]
        + [pltpu.VMEM((CHUNK, HD), F32) for _ in range(2)]
        + [pltpu.VMEM_SHARED((NP, HD), F32)]
        + [pltpu.SemaphoreType.DMA for _ in range(8)]
    ),
)


def _mm_bias_body(x_ref, w_ref, b_ref, o_ref):
    o_ref[...] = (jnp.dot(x_ref[...], w_ref[...], preferred_element_type=F32)
                  + b_ref[...])


def _mm_bias(x, w, b, bm):
    m, k = x.shape
    hd = w.shape[1]
    return pl.pallas_call(
        _mm_bias_body,
        grid=(m // bm,),
        in_specs=[
            pl.BlockSpec((bm, k), lambda i: (i, 0)),
            pl.BlockSpec((k, hd), lambda i: (0, 0)),
            pl.BlockSpec((1, hd), lambda i: (0, 0)),
        ],
        out_specs=pl.BlockSpec((bm, hd), lambda i: (i, 0)),
        out_shape=jax.ShapeDtypeStruct((m, hd), F32),
    )(x, w, b.reshape(1, hd))


BM = 2048


def _onehot(b_ref):
    ids = jax.lax.broadcasted_iota(jnp.int32, (BM, G), 1)
    return (b_ref[...] == ids).astype(F32)


def _update_pool_body(p0, p1, y, wua, wub, bu, b_ref, o_y, o_pool):
    agg = p0[...] + p1[...]
    yn = jnp.maximum(
        jnp.dot(agg, wua[...], preferred_element_type=F32)
        + jnp.dot(y[...], wub[...], preferred_element_type=F32)
        + bu[...], 0.0)
    o_y[...] = yn

    @pl.when(pl.program_id(0) == 0)
    def _():
        o_pool[...] = jnp.zeros_like(o_pool)
    oh = _onehot(b_ref)
    o_pool[...] += jax.lax.dot_general(
        oh, yn, (((0,), (0,)), ((), ())), preferred_element_type=F32,
        precision=jax.lax.Precision.HIGHEST)


def _update_pool(p0, p1, y, wua, wub, bu, bidx):
    return pl.pallas_call(
        _update_pool_body,
        grid=(NP // BM,),
        in_specs=[
            pl.BlockSpec((BM, HD), lambda i: (i, 0)),
            pl.BlockSpec((BM, HD), lambda i: (i, 0)),
            pl.BlockSpec((BM, HD), lambda i: (i, 0)),
            pl.BlockSpec((HD, HD), lambda i: (0, 0)),
            pl.BlockSpec((HD, HD), lambda i: (0, 0)),
            pl.BlockSpec((1, HD), lambda i: (0, 0)),
            pl.BlockSpec((BM, 1), lambda i: (i, 0)),
        ],
        out_specs=[
            pl.BlockSpec((BM, HD), lambda i: (i, 0)),
            pl.BlockSpec((G, HD), lambda i: (0, 0)),
        ],
        out_shape=[
            jax.ShapeDtypeStruct((NP, HD), F32),
            jax.ShapeDtypeStruct((G, HD), F32),
        ],
    )(p0, p1, y, wua, wub, bu.reshape(1, HD), bidx)


def _vn_z_body(y, pool, wv, bv, b_ref, wma, o_y2, o_z):
    v = jnp.maximum(
        jnp.dot(pool[...], wv[...], preferred_element_type=F32) + bv[...], 0.0)
    oh = _onehot(b_ref)
    y2 = y[...] + jnp.dot(oh, v, preferred_element_type=F32,
                          precision=jax.lax.Precision.HIGHEST)
    o_y2[...] = y2
    o_z[...] = jnp.dot(y2, wma[...], preferred_element_type=F32)


def _vn_z(y, pool, wv, bv, bidx, wma):
    return pl.pallas_call(
        _vn_z_body,
        grid=(NP // BM,),
        in_specs=[
            pl.BlockSpec((BM, HD), lambda i: (i, 0)),
            pl.BlockSpec((G, HD), lambda i: (0, 0)),
            pl.BlockSpec((HD, HD), lambda i: (0, 0)),
            pl.BlockSpec((1, HD), lambda i: (0, 0)),
            pl.BlockSpec((BM, 1), lambda i: (i, 0)),
            pl.BlockSpec((HD, HD), lambda i: (0, 0)),
        ],
        out_specs=[
            pl.BlockSpec((BM, HD), lambda i: (i, 0)),
            pl.BlockSpec((BM, HD), lambda i: (i, 0)),
        ],
        out_shape=[
            jax.ShapeDtypeStruct((NP, HD), F32),
            jax.ShapeDtypeStruct((NP, HD), F32),
        ],
    )(y, pool, wv, bv.reshape(1, HD), bidx, wma)


def _head_body(pool, wout, bout, o_ref):
    o_ref[...] = (jnp.dot(pool[...], wout[...], preferred_element_type=F32)
                  + bout[...])


def _head(pool, wout, bout):
    return pl.pallas_call(
        _head_body,
        grid=(1,),
        in_specs=[
            pl.BlockSpec((G, HD), lambda i: (0, 0)),
            pl.BlockSpec((HD, 1), lambda i: (0, 0)),
            pl.BlockSpec((1, 1), lambda i: (0, 0)),
        ],
        out_specs=pl.BlockSpec((G, 1), lambda i: (0, 0)),
        out_shape=jax.ShapeDtypeStruct((G, 1), F32),
    )(pool, wout, bout.reshape(1, 1))


def kernel(H, Xe, id_Xe, batch_idx, params):
    padE = EP - E
    src = jnp.concatenate([id_Xe[0], jnp.zeros((padE,), jnp.int32)])
    dst = jnp.concatenate(
        [id_Xe[1], N + (jnp.arange(padE, dtype=jnp.int32) % (NP - N))])
    idx2 = jnp.stack([src.reshape(NW, NCHK, CHUNK),
                      dst.reshape(NW, NCHK, CHUNK)], axis=2)
    Xep = jnp.pad(Xe, ((0, padE), (0, 0)))
    Hp = jnp.pad(H, ((0, NP - N), (0, 0)))
    bidx = jnp.pad(batch_idx, (0, NP - N), constant_values=G).reshape(NP, 1)

    p = params
    Wm = [p['Wm0'], p['Wm1'], p['Wm2']]
    bm = [p['bm0'], p['bm1'], p['bm2']]
    Wu = [p['Wu0'], p['Wu1'], p['Wu2']]
    bu = [p['bu0'], p['bu1'], p['bu2']]
    Wv = [p['Wv0'], p['Wv1']]
    bv = [p['bv0'], p['bv1']]

    C = [_mm_bias(Xep, Wm[l][DF:], bm[l], 2048) for l in range(3)]

    y = Hp
    Z = _mm_bias(Hp, Wm[0][:DF], jnp.zeros((HD,), F32), BM)
    pool = None
    for l in range(3):
        P = _sc_edge_pass(Z, C[l], idx2)
        y, pool = _update_pool(P[0], P[1], y, Wu[l][:HD], Wu[l][HD:],
                               bu[l], bidx)
        if l < 2:
            y, Z = _vn_z(y, pool, Wv[l], bv[l], bidx, Wm[l + 1][:DF])

    return _head(pool, p['Wout'], p['bout'])


# speedup vs baseline: 1.5463x; 1.5463x over previous
import jax
import jax.numpy as jnp
from jax import lax
from jax.experimental import pallas as pl
from jax.experimental.pallas import tpu as pltpu
from jax.experimental.pallas import tpu_sc as plsc

N = 10000
E = 320000
DF = 128
DE = 16
HD = 128
G = 64

NP = 10240
NC, NS, L = 2, 16, 16
NW = NC * NS
CHUNK = 96
NCHK = 105
EPT = CHUNK * NCHK
EP = NW * EPT
RPT = NP // NS
F32 = jnp.float32


def _sc_edge_body(z_hbm, c_hbm, idx_hbm, out_hbm,
                  idxA, idxB, idxC, bufA, bufB, bufC, agg_sh,
                  isA, isB, isC, csA, csB, csC,
                  gsA, gsB, gsC, ssA, ssB, ssC):
    c = lax.axis_index("c")
    s = lax.axis_index("s")
    tile = c * NS + s
    ebase = tile * EPT
    idxs = (idxA, idxB, idxC)
    bufs = (bufA, bufB, bufC)
    iss = (isA, isB, isC)
    css = (csA, csB, csC)
    gss = (gsA, gsB, gsC)
    sss = (ssA, ssB, ssC)

    def zset(i, carry):
        for k in range(HD // L):
            bufA[i, pl.ds(k * L, L)] = jnp.zeros((L,), F32)
        return carry
    lax.fori_loop(0, CHUNK, zset, 0)
    r0 = s * RPT
    nfull = RPT // CHUNK
    rem = RPT - nfull * CHUNK
    for q in range(nfull):
        pltpu.sync_copy(bufA, agg_sh.at[pl.ds(r0 + q * CHUNK, CHUNK)])
    if rem:
        pltpu.sync_copy(bufA.at[pl.ds(0, rem)],
                        agg_sh.at[pl.ds(r0 + nfull * CHUNK, rem)])
    plsc.subcore_barrier()

    def ixissue(j, b):
        pltpu.async_copy(idx_hbm.at[tile, j], idxs[b], iss[b])

    def ixwait(b):
        pltpu.make_async_copy(idx_hbm.at[tile, 0], idxs[b], iss[b]).wait()

    def cissue(j, b):
        pltpu.async_copy(c_hbm.at[pl.ds(ebase + j * CHUNK, CHUNK)],
                         bufs[b], css[b])

    def cwait(b):
        pltpu.make_async_copy(c_hbm.at[pl.ds(ebase, CHUNK)],
                              bufs[b], css[b]).wait()

    def gissue(b):
        pltpu.async_copy(z_hbm.at[idxs[b].at[0]], bufs[b], gss[b], add=True)

    def gwait(b):
        pltpu.make_async_copy(z_hbm.at[idxs[b].at[0]], bufs[b], gss[b]).wait()

    def sissue(b):
        pltpu.async_copy(bufs[b], agg_sh.at[idxs[b].at[1]], sss[b], add=True)

    def swait(b):
        pltpu.make_async_copy(bufs[b], agg_sh.at[idxs[b].at[1]], sss[b]).wait()

    def relu_buf(buf):
        def erow(e, cc):
            for k in range(HD // L):
                sl = pl.ds(k * L, L)
                buf[e, sl] = jnp.maximum(buf[e, sl], 0.0)
            return cc
        lax.fori_loop(0, CHUNK, erow, 0)

    TRIPS = NCHK // 3
    assert NCHK == 3 * TRIPS
    ixissue(0, 0)
    cissue(0, 0)
    ixissue(1, 1)
    cissue(1, 1)
    ixwait(0)
    cwait(0)
    gissue(0)

    def trip_body(i, carry):
        for b in (0, 1, 2):
            j = 3 * i + b
            n1 = (b + 1) % 3
            n2 = (b + 2) % 3
            gwait(b)
            relu_buf(bufs[b])
            sissue(b)

            @pl.when(j >= 1)
            def _():
                swait(n2)

            @pl.when(j + 2 < NCHK)
            def _():
                ixissue(j + 2, n2)
                cissue(j + 2, n2)

            @pl.when(j + 1 < NCHK)
            def _():
                ixwait(n1)
                cwait(n1)
                gissue(n1)
        return carry
    lax.fori_loop(0, TRIPS, trip_body, 0)

    swait((NCHK - 1) % 3)
    plsc.subcore_barrier()

    for q in range(nfull):
        rr = r0 + q * CHUNK
        pltpu.sync_copy(agg_sh.at[pl.ds(rr, CHUNK)], bufA)
        pltpu.sync_copy(bufA, out_hbm.at[c, pl.ds(rr, CHUNK)])
    if rem:
        rr = r0 + nfull * CHUNK
        pltpu.sync_copy(agg_sh.at[pl.ds(rr, rem)], bufA.at[pl.ds(0, rem)])
        pltpu.sync_copy(bufA.at[pl.ds(0, rem)], out_hbm.at[c, pl.ds(rr, rem)])


_sc_edge_pass = pl.kernel(
    _sc_edge_body,
    out_type=jax.ShapeDtypeStruct((NC, NP, HD), F32),
    mesh=plsc.VectorSubcoreMesh(core_axis_name="c", subcore_axis_name="s",
                                num_cores=NC, num_subcores=NS),
    scratch_types=(
        [pltpu.VMEM((2, CHUNK), jnp.int32) for _ in range(3)]
        + [pltpu.VMEM((CHUNK, HD), F32) for _ in range(3)]
        + [pltpu.VMEM_SHARED((NP, HD), F32)]
        + [pltpu.SemaphoreType.DMA for _ in range(12)]
    ),
)


def _mm_bias_body(x_ref, w_ref, b_ref, o_ref):
    o_ref[...] = (jnp.dot(x_ref[...], w_ref[...], preferred_element_type=F32)
                  + b_ref[...])


def _mm_bias(x, w, b, bm):
    m, k = x.shape
    hd = w.shape[1]
    return pl.pallas_call(
        _mm_bias_body,
        grid=(m // bm,),
        in_specs=[
            pl.BlockSpec((bm, k), lambda i: (i, 0)),
            pl.BlockSpec((k, hd), lambda i: (0, 0)),
            pl.BlockSpec((1, hd), lambda i: (0, 0)),
        ],
        out_specs=pl.BlockSpec((bm, hd), lambda i: (i, 0)),
        out_shape=jax.ShapeDtypeStruct((m, hd), F32),
    )(x, w, b.reshape(1, hd))


BM = 2048


def _onehot(b_ref):
    ids = jax.lax.broadcasted_iota(jnp.int32, (BM, G), 1)
    return (b_ref[...] == ids).astype(F32)


def _update_pool_body(p0, p1, y, wua, wub, bu, b_ref, o_y, o_pool):
    agg = p0[...] + p1[...]
    yn = jnp.maximum(
        jnp.dot(agg, wua[...], preferred_element_type=F32)
        + jnp.dot(y[...], wub[...], preferred_element_type=F32)
        + bu[...], 0.0)
    o_y[...] = yn

    @pl.when(pl.program_id(0) == 0)
    def _():
        o_pool[...] = jnp.zeros_like(o_pool)
    oh = _onehot(b_ref)
    o_pool[...] += jax.lax.dot_general(
        oh, yn, (((0,), (0,)), ((), ())), preferred_element_type=F32,
        precision=jax.lax.Precision.HIGHEST)


def _update_pool(p0, p1, y, wua, wub, bu, bidx):
    return pl.pallas_call(
        _update_pool_body,
        grid=(NP // BM,),
        in_specs=[
            pl.BlockSpec((BM, HD), lambda i: (i, 0)),
            pl.BlockSpec((BM, HD), lambda i: (i, 0)),
            pl.BlockSpec((BM, HD), lambda i: (i, 0)),
            pl.BlockSpec((HD, HD), lambda i: (0, 0)),
            pl.BlockSpec((HD, HD), lambda i: (0, 0)),
            pl.BlockSpec((1, HD), lambda i: (0, 0)),
            pl.BlockSpec((BM, 1), lambda i: (i, 0)),
        ],
        out_specs=[
            pl.BlockSpec((BM, HD), lambda i: (i, 0)),
            pl.BlockSpec((G, HD), lambda i: (0, 0)),
        ],
        out_shape=[
            jax.ShapeDtypeStruct((NP, HD), F32),
            jax.ShapeDtypeStruct((G, HD), F32),
        ],
    )(p0, p1, y, wua, wub, bu.reshape(1, HD), bidx)


def _vn_z_body(y, pool, wv, bv, b_ref, wma, o_y2, o_z):
    v = jnp.maximum(
        jnp.dot(pool[...], wv[...], preferred_element_type=F32) + bv[...], 0.0)
    oh = _onehot(b_ref)
    y2 = y[...] + jnp.dot(oh, v, preferred_element_type=F32,
                          precision=jax.lax.Precision.HIGHEST)
    o_y2[...] = y2
    o_z[...] = jnp.dot(y2, wma[...], preferred_element_type=F32)


def _vn_z(y, pool, wv, bv, bidx, wma):
    return pl.pallas_call(
        _vn_z_body,
        grid=(NP // BM,),
        in_specs=[
            pl.BlockSpec((BM, HD), lambda i: (i, 0)),
            pl.BlockSpec((G, HD), lambda i: (0, 0)),
            pl.BlockSpec((HD, HD), lambda i: (0, 0)),
            pl.BlockSpec((1, HD), lambda i: (0, 0)),
            pl.BlockSpec((BM, 1), lambda i: (i, 0)),
            pl.BlockSpec((HD, HD), lambda i: (0, 0)),
        ],
        out_specs=[
            pl.BlockSpec((BM, HD), lambda i: (i, 0)),
            pl.BlockSpec((BM, HD), lambda i: (i, 0)),
        ],
        out_shape=[
            jax.ShapeDtypeStruct((NP, HD), F32),
            jax.ShapeDtypeStruct((NP, HD), F32),
        ],
    )(y, pool, wv, bv.reshape(1, HD), bidx, wma)


def _head_body(pool, wout, bout, o_ref):
    o_ref[...] = (jnp.dot(pool[...], wout[...], preferred_element_type=F32)
                  + bout[...])


def _head(pool, wout, bout):
    return pl.pallas_call(
        _head_body,
        grid=(1,),
        in_specs=[
            pl.BlockSpec((G, HD), lambda i: (0, 0)),
            pl.BlockSpec((HD, 1), lambda i: (0, 0)),
            pl.BlockSpec((1, 1), lambda i: (0, 0)),
        ],
        out_specs=pl.BlockSpec((G, 1), lambda i: (0, 0)),
        out_shape=jax.ShapeDtypeStruct((G, 1), F32),
    )(pool, wout, bout.reshape(1, 1))


def kernel(H, Xe, id_Xe, batch_idx, params):
    padE = EP - E
    src = jnp.concatenate([id_Xe[0], jnp.zeros((padE,), jnp.int32)])
    dst = jnp.concatenate(
        [id_Xe[1], N + (jnp.arange(padE, dtype=jnp.int32) % (NP - N))])
    idx2 = jnp.stack([src.reshape(NW, NCHK, CHUNK),
                      dst.reshape(NW, NCHK, CHUNK)], axis=2)
    Xep = jnp.pad(Xe, ((0, padE), (0, 0)))
    Hp = jnp.pad(H, ((0, NP - N), (0, 0)))
    bidx = jnp.pad(batch_idx, (0, NP - N), constant_values=G).reshape(NP, 1)

    p = params
    Wm = [p['Wm0'], p['Wm1'], p['Wm2']]
    bm = [p['bm0'], p['bm1'], p['bm2']]
    Wu = [p['Wu0'], p['Wu1'], p['Wu2']]
    bu = [p['bu0'], p['bu1'], p['bu2']]
    Wv = [p['Wv0'], p['Wv1']]
    bv = [p['bv0'], p['bv1']]

    C = [_mm_bias(Xep, Wm[l][DF:], bm[l], 2520) for l in range(3)]

    y = Hp
    Z = _mm_bias(Hp, Wm[0][:DF], jnp.zeros((HD,), F32), BM)
    pool = None
    for l in range(3):
        P = _sc_edge_pass(Z, C[l], idx2)
        y, pool = _update_pool(P[0], P[1], y, Wu[l][:HD], Wu[l][HD:],
                               bu[l], bidx)
        if l < 2:
            y, Z = _vn_z(y, pool, Wv[l], bv[l], bidx, Wm[l + 1][:DF])

    return _head(pool, p['Wout'], p['bout'])
